# baseline (speedup 1.0000x reference)
import jax
import jax.numpy as jnp
from jax.experimental import pallas as pl
from jax.experimental.pallas import tpu as pltpu

_N = 16384
_D = 1024
_CH = 4096
_NCHUNK = _N // _CH
_NBUF = 3


def _body(x_hbm, o_hbm, *rest):
    bufs = rest[:_NBUF]
    insem, outsem = rest[_NBUF], rest[_NBUF + 1]

    def in_copy(k):
        return pltpu.make_async_copy(
            x_hbm.at[pl.ds(k * _CH, _CH)], bufs[k % _NBUF], insem.at[k % _NBUF])

    def out_copy(k):
        return pltpu.make_async_copy(
            bufs[k % _NBUF], o_hbm.at[pl.ds(k * _CH, _CH)], outsem.at[k % _NBUF])

    for k in range(_NBUF):
        in_copy(k).start()
    for k in range(_NCHUNK):
        in_copy(k).wait()
        buf = bufs[k % _NBUF]
        row_col = (jax.lax.broadcasted_iota(jnp.int32, (_CH, 1), 0)
                   + k * _CH).astype(jnp.float32)
        buf[...] = buf[...] * 2.0 + row_col
        out_copy(k).start()
        j = k - 1
        if j >= 0 and j + _NBUF < _NCHUNK:
            out_copy(j).wait()
            in_copy(j + _NBUF).start()
    for k in range(_NCHUNK - _NBUF, _NCHUNK):
        out_copy(k).wait()


def kernel(input_tensor):
    return pl.pallas_call(
        _body,
        in_specs=[pl.BlockSpec(memory_space=pl.ANY)],
        out_specs=pl.BlockSpec(memory_space=pl.ANY),
        out_shape=jax.ShapeDtypeStruct((_N, _D), input_tensor.dtype),
        scratch_shapes=(
            [pltpu.VMEM((_CH, _D), jnp.float32) for _ in range(_NBUF)]
            + [pltpu.SemaphoreType.DMA((_NBUF,)),
               pltpu.SemaphoreType.DMA((_NBUF,))]
        ),
        compiler_params=pltpu.CompilerParams(
            vmem_limit_bytes=64 * 1024 * 1024,
        ),
    )(input_tensor)


# speedup vs baseline: 1.0223x; 1.0223x over previous
import jax
import jax.numpy as jnp
from jax.experimental import pallas as pl
from jax.experimental.pallas import tpu as pltpu

_N = 16384
_D = 1024
_CH = 4096
_NCHUNK = _N // _CH
_NBUF = 3


def _body(x_hbm, o_hbm, *rest):
    bufs = rest[:_NBUF]
    insem, outsem = rest[_NBUF], rest[_NBUF + 1]

    def in_copy(k):
        return pltpu.make_async_copy(
            x_hbm.at[pl.ds(k * _CH, _CH)], bufs[k % _NBUF], insem.at[k % _NBUF])

    def out_copy(k):
        return pltpu.make_async_copy(
            bufs[k % _NBUF], o_hbm.at[pl.ds(k * _CH, _CH)], outsem.at[k % _NBUF])

    for k in range(_NBUF):
        in_copy(k).start()
    for k in range(_NCHUNK):
        in_copy(k).wait()
        buf = bufs[k % _NBUF]
        row_col = (jax.lax.broadcasted_iota(jnp.int32, (_CH, 1), 0)
                   + k * _CH).astype(jnp.float32)
        buf[...] = buf[...] * 2.0 + row_col
        out_copy(k).start()
        if k + _NBUF < _NCHUNK:
            out_copy(k).wait()
            in_copy(k + _NBUF).start()
    for k in range(_NCHUNK - _NBUF, _NCHUNK):
        out_copy(k).wait()


def kernel(input_tensor):
    return pl.pallas_call(
        _body,
        in_specs=[pl.BlockSpec(memory_space=pl.ANY)],
        out_specs=pl.BlockSpec(memory_space=pl.ANY),
        out_shape=jax.ShapeDtypeStruct((_N, _D), input_tensor.dtype),
        scratch_shapes=(
            [pltpu.VMEM((_CH, _D), jnp.float32) for _ in range(_NBUF)]
            + [pltpu.SemaphoreType.DMA((_NBUF,)),
               pltpu.SemaphoreType.DMA((_NBUF,))]
        ),
        compiler_params=pltpu.CompilerParams(
            vmem_limit_bytes=64 * 1024 * 1024,
        ),
    )(input_tensor)
